# baseline (speedup 1.0000x reference)
import functools

import jax
import jax.numpy as jnp
from jax import lax
from jax.experimental import pallas as pl
from jax.experimental.pallas import tpu as pltpu
from jax.experimental.pallas import tpu_sc as plsc

_NUM_UNITS = 500
_SPEECH_DIM = 256
_HIDDEN = 1024
_LLM_DIM = 2048
_BB = 16

_INV_SQRT2 = 0.7071067811865476

_info = plsc.get_sparse_core_info()
_NC, _NS = _info.num_cores, _info.num_subcores
_NW = _NC * _NS
_CH = 80


def _make_sc_gather(n):
    rows_per_w = n // _NW
    n_chunks = rows_per_w // _CH
    assert rows_per_w % _CH == 0
    mesh = plsc.VectorSubcoreMesh(core_axis_name="c", subcore_axis_name="s")

    @functools.partial(
        pl.kernel, mesh=mesh,
        out_type=jax.ShapeDtypeStruct((n, _SPEECH_DIM), jnp.float32),
        scratch_types=[
            pltpu.VMEM((_CH,), jnp.int32),
            pltpu.VMEM((_CH,), jnp.int32),
            pltpu.VMEM((_CH, _SPEECH_DIM), jnp.float32),
            pltpu.VMEM((_CH, _SPEECH_DIM), jnp.float32),
            pltpu.SemaphoreType.DMA,
            pltpu.SemaphoreType.DMA,
            pltpu.SemaphoreType.DMA,
            pltpu.SemaphoreType.DMA,
        ],
    )
    def sc_gather(idx_hbm, table_hbm, out_hbm, idx_v0, idx_v1, rows_v0,
                  rows_v1, sg0, sg1, ss0, ss1):
        idx_v = (idx_v0, idx_v1)
        rows_v = (rows_v0, rows_v1)
        sg = (sg0, sg1)
        ss = (ss0, ss1)
        wid = lax.axis_index("s") * _NC + lax.axis_index("c")
        base0 = wid * rows_per_w

        pltpu.sync_copy(idx_hbm.at[pl.ds(base0, _CH)], idx_v[0])
        g = pltpu.async_copy(table_hbm.at[idx_v[0]], rows_v[0], sg[0])
        scatters = [None, None]
        for c in range(n_chunks):
            p = c & 1
            q = p ^ 1
            if c + 1 < n_chunks:
                pltpu.sync_copy(
                    idx_hbm.at[pl.ds(base0 + (c + 1) * _CH, _CH)], idx_v[q])
            g.wait()
            scatters[p] = pltpu.async_copy(
                rows_v[p], out_hbm.at[pl.ds(base0 + c * _CH, _CH)], ss[p])
            if c + 1 < n_chunks:
                if scatters[q] is not None:
                    scatters[q].wait()
                g = pltpu.async_copy(table_hbm.at[idx_v[q]], rows_v[q], sg[q])
        for s in scatters:
            if s is not None:
                s.wait()

    return sc_gather


def _mlp_ln(x_ref, w1_ref, b1_ref, w2_ref, b2_ref, g_ref, bt_ref, o_ref):
    x = x_ref[...]
    h = jnp.dot(x, w1_ref[...], preferred_element_type=jnp.float32)
    h = h + b1_ref[...]
    h = 0.5 * h * (1.0 + lax.erf(h * _INV_SQRT2))
    y = jnp.dot(h, w2_ref[...], preferred_element_type=jnp.float32)
    y = y + b2_ref[...]
    mu = jnp.mean(y, axis=-1, keepdims=True)
    yc = y - mu
    var = jnp.mean(yc * yc, axis=-1, keepdims=True)
    out = yc * lax.rsqrt(var + 1e-5) * g_ref[...] + bt_ref[...]
    o_ref[...] = out.reshape(o_ref.shape)


def _mlp_ln_alias(x_ref, w1_ref, b1_ref, w2_ref, b2_ref, g_ref, bt_ref,
                  prev_ref, o_ref):
    del prev_ref
    _mlp_ln(x_ref, w1_ref, b1_ref, w2_ref, b2_ref, g_ref, bt_ref, o_ref)


@jax.jit
def kernel(local_ids, embed_W, W1, b1, W2, b2, ln_gamma, ln_beta):
    B, T = local_ids.shape
    n = B * T
    grid = B // _BB
    rows = T * _BB
    half_g = grid // 2
    nh = n // 2
    ids = (local_ids.astype(jnp.int32).T
           .reshape(T, grid, _BB).transpose(1, 0, 2).reshape(n))
    x = _make_sc_gather(n)(ids, embed_W)

    full = lambda *shape: pl.BlockSpec(shape, lambda i: (0,) * len(shape))
    w_specs = [
        full(_SPEECH_DIM, _HIDDEN),
        full(1, _HIDDEN),
        full(_HIDDEN, _LLM_DIM),
        full(1, _LLM_DIM),
        full(1, _LLM_DIM),
        full(1, _LLM_DIM),
    ]
    w_args = (W1, b1.reshape(1, _HIDDEN), W2, b2.reshape(1, _LLM_DIM),
              ln_gamma.reshape(1, _LLM_DIM), ln_beta.reshape(1, _LLM_DIM))
    x_spec = pl.BlockSpec((rows, _SPEECH_DIM), lambda i: (i, 0))
    out_shape = jax.ShapeDtypeStruct((T, B, _LLM_DIM), jnp.float32)
    params = pltpu.CompilerParams(dimension_semantics=("arbitrary",))

    out = pl.pallas_call(
        _mlp_ln,
        grid=(grid,),
        in_specs=[x_spec] + w_specs,
        out_specs=pl.BlockSpec((T, _BB, _LLM_DIM), lambda i: (0, i, 0)),
        out_shape=out_shape,
        compiler_params=params,
    )(x, *w_args)
    return out.transpose(1, 0, 2)


# speedup vs baseline: 1.0577x; 1.0577x over previous
import functools

import jax
import jax.numpy as jnp
from jax import lax
from jax.experimental import pallas as pl
from jax.experimental.pallas import tpu as pltpu
from jax.experimental.pallas import tpu_sc as plsc

_NUM_UNITS = 500
_SPEECH_DIM = 256
_HIDDEN = 1024
_LLM_DIM = 2048
_BB = 16

_INV_SQRT2 = 0.7071067811865476

_info = plsc.get_sparse_core_info()
_NC, _NS = _info.num_cores, _info.num_subcores
_NW = _NC * _NS
_CH = 80


def _make_sc_gather(n):
    rows_per_w = n // _NW
    n_chunks = rows_per_w // _CH
    assert rows_per_w % _CH == 0
    mesh = plsc.VectorSubcoreMesh(core_axis_name="c", subcore_axis_name="s")

    @functools.partial(
        pl.kernel, mesh=mesh,
        out_type=jax.ShapeDtypeStruct((n, _SPEECH_DIM), jnp.float32),
        scratch_types=[
            pltpu.VMEM((_CH,), jnp.int32),
            pltpu.VMEM((_CH,), jnp.int32),
            pltpu.VMEM((_CH, _SPEECH_DIM), jnp.float32),
            pltpu.VMEM((_CH, _SPEECH_DIM), jnp.float32),
            pltpu.SemaphoreType.DMA,
            pltpu.SemaphoreType.DMA,
            pltpu.SemaphoreType.DMA,
            pltpu.SemaphoreType.DMA,
        ],
    )
    def sc_gather(idx_hbm, table_hbm, out_hbm, idx_v0, idx_v1, rows_v0,
                  rows_v1, sg0, sg1, ss0, ss1):
        idx_v = (idx_v0, idx_v1)
        rows_v = (rows_v0, rows_v1)
        sg = (sg0, sg1)
        ss = (ss0, ss1)
        wid = lax.axis_index("s") * _NC + lax.axis_index("c")
        base0 = wid * rows_per_w

        pltpu.sync_copy(idx_hbm.at[pl.ds(base0, _CH)], idx_v[0])
        g = pltpu.async_copy(table_hbm.at[idx_v[0]], rows_v[0], sg[0])
        scatters = [None, None]
        for c in range(n_chunks):
            p = c & 1
            q = p ^ 1
            if c + 1 < n_chunks:
                pltpu.sync_copy(
                    idx_hbm.at[pl.ds(base0 + (c + 1) * _CH, _CH)], idx_v[q])
            g.wait()
            scatters[p] = pltpu.async_copy(
                rows_v[p], out_hbm.at[pl.ds(base0 + c * _CH, _CH)], ss[p])
            if c + 1 < n_chunks:
                if scatters[q] is not None:
                    scatters[q].wait()
                g = pltpu.async_copy(table_hbm.at[idx_v[q]], rows_v[q], sg[q])
        for s in scatters:
            if s is not None:
                s.wait()

    return sc_gather


def _mlp_ln(x_ref, w1_ref, b1_ref, w2_ref, b2_ref, g_ref, bt_ref, o_ref):
    x = x_ref[...]
    h = jnp.dot(x, w1_ref[...], preferred_element_type=jnp.float32)
    h = h + b1_ref[...]
    h = 0.5 * h * (1.0 + lax.erf(h * _INV_SQRT2))
    y = jnp.dot(h, w2_ref[...], preferred_element_type=jnp.float32)
    y = y + b2_ref[...]
    mu = jnp.mean(y, axis=-1, keepdims=True)
    yc = y - mu
    var = jnp.mean(yc * yc, axis=-1, keepdims=True)
    out = yc * lax.rsqrt(var + 1e-5) * g_ref[...] + bt_ref[...]
    o_ref[...] = out.reshape(o_ref.shape)


def _mlp_ln_alias(x_ref, w1_ref, b1_ref, w2_ref, b2_ref, g_ref, bt_ref,
                  prev_ref, o_ref):
    del prev_ref
    _mlp_ln(x_ref, w1_ref, b1_ref, w2_ref, b2_ref, g_ref, bt_ref, o_ref)


@jax.jit
def kernel(local_ids, embed_W, W1, b1, W2, b2, ln_gamma, ln_beta):
    B, T = local_ids.shape
    n = B * T
    grid = B // _BB
    rows = T * _BB
    half_g = grid // 2
    nh = n // 2
    ids = (local_ids.astype(jnp.int32).T
           .reshape(T, grid, _BB).transpose(1, 0, 2).reshape(n))
    gather = _make_sc_gather(nh)
    x0 = gather(ids[:nh], embed_W)
    x1 = gather(ids[nh:], embed_W)

    full = lambda *shape: pl.BlockSpec(shape, lambda i: (0,) * len(shape))
    w_specs = [
        full(_SPEECH_DIM, _HIDDEN),
        full(1, _HIDDEN),
        full(_HIDDEN, _LLM_DIM),
        full(1, _LLM_DIM),
        full(1, _LLM_DIM),
        full(1, _LLM_DIM),
    ]
    w_args = (W1, b1.reshape(1, _HIDDEN), W2, b2.reshape(1, _LLM_DIM),
              ln_gamma.reshape(1, _LLM_DIM), ln_beta.reshape(1, _LLM_DIM))
    x_spec = pl.BlockSpec((rows, _SPEECH_DIM), lambda i: (i, 0))
    out_shape = jax.ShapeDtypeStruct((T, B, _LLM_DIM), jnp.float32)
    params = pltpu.CompilerParams(dimension_semantics=("arbitrary",))

    out0 = pl.pallas_call(
        _mlp_ln,
        grid=(half_g,),
        in_specs=[x_spec] + w_specs,
        out_specs=pl.BlockSpec((T, _BB, _LLM_DIM), lambda i: (0, i, 0)),
        out_shape=out_shape,
        compiler_params=params,
    )(x0, *w_args)
    out = pl.pallas_call(
        _mlp_ln_alias,
        grid=(half_g,),
        in_specs=[x_spec] + w_specs + [
            pl.BlockSpec(memory_space=pl.ANY)],
        out_specs=pl.BlockSpec((T, _BB, _LLM_DIM),
                               lambda i: (0, i + half_g, 0)),
        out_shape=out_shape,
        input_output_aliases={7: 0},
        compiler_params=params,
    )(x1, *w_args, out0)
    return out.transpose(1, 0, 2)
